# Initial kernel scaffold; baseline (speedup 1.0000x reference)
import jax
import jax.numpy as jnp
from jax.experimental import pallas as pl


def kernel(x, W1a, b1a, coeffs, W3a, b3a, g1a, be1a, g2a, be2a, g3a, be3a, G_rows, G_cols, G_vals, L_rows, L_cols, L_vals, F_rows, F_cols, F_vals, EW, NS):
    raise NotImplementedError("write your pallas kernel here")


# speedup vs baseline: 62.7376x; 62.7376x over previous
import functools

import jax
import jax.numpy as jnp
from jax import lax
from jax.experimental import pallas as pl
from jax.experimental.pallas import tpu as pltpu
from jax.experimental.pallas import tpu_sc as plsc

NV = 40962
NF = 81920
B = 4
IN_CH = 32
NECK = 16
OUT_CH = 32
EPS = 1e-5

NVP = 43008
TR = 1024
CH = 8
OT = 64
NC = 2
NS = 16
NW = NC * NS

f32 = jnp.float32
i32 = jnp.int32


def _make_sc_gather(t_rows, din, r_rows, k_nnz, n_w):
    dout = din * n_w
    rows_pw = r_rows // NW
    chunks_pw = rows_pw // CH
    tiles_pw = rows_pw // OT
    chunks_pt = OT // CH
    mesh = plsc.VectorSubcoreMesh(core_axis_name="c", subcore_axis_name="s")

    def body(table, idxh, wh, outh, idx_v, w_v, rows_v, out_v, gsem):
        cid = lax.axis_index("c")
        sid = lax.axis_index("s")
        wid = sid * NC + cid
        cbase = wid * chunks_pw
        pltpu.sync_copy(idxh.at[pl.ds(cbase, chunks_pw)], idx_v)
        pltpu.sync_copy(wh.at[pl.ds(cbase, chunks_pw)], w_v)

        rowlen = CH * k_nnz * n_w
        offs = list(range(0, max(rowlen - 15, 1), 16))
        if rowlen % 16:
            offs.append(rowlen - 16)
        nv = din // 16

        def tile_body(t, _):
            def chunk_body(c, _):
                cl = t * chunks_pt + c
                pltpu.async_copy(table.at[idx_v.at[cl]], rows_v, gsem).wait()
                wvecs = [w_v[cl, pl.ds(o, 16)] for o in offs]

                def wscal(j):
                    if j >= offs[-1]:
                        return wvecs[-1][j - offs[-1]]
                    return wvecs[j // 16][j % 16]

                for r in range(CH):
                    accs = [[None] * nv for _ in range(n_w)]
                    for kk in range(k_nnz):
                        ws = [wscal((r * k_nnz + kk) * n_w + w) for w in range(n_w)]
                        for v in range(nv):
                            rv = rows_v[r * k_nnz + kk, pl.ds(v * 16, 16)]
                            for w in range(n_w):
                                p = ws[w] * rv
                                accs[w][v] = p if kk == 0 else accs[w][v] + p
                    for w in range(n_w):
                        for v in range(nv):
                            out_v[c * CH + r,
                                  pl.ds(w * din + v * 16, 16)] = accs[w][v]
                return 0

            lax.fori_loop(0, chunks_pt, chunk_body, 0)
            pltpu.sync_copy(out_v, outh.at[pl.ds(wid * rows_pw + t * OT, OT)])
            return 0

        lax.fori_loop(0, tiles_pw, tile_body, 0)

    return functools.partial(
        pl.kernel,
        out_type=jax.ShapeDtypeStruct((r_rows, dout), f32),
        mesh=mesh,
        scratch_types=[
            pltpu.VMEM((chunks_pw, CH * k_nnz), i32),
            pltpu.VMEM((chunks_pw, CH * k_nnz * n_w), f32),
            pltpu.VMEM((CH * k_nnz, din), f32),
            pltpu.VMEM((OT, dout), f32),
            pltpu.SemaphoreType.DMA,
        ],
        compiler_params=pltpu.CompilerParams(use_tc_tiling_on_sc=False),
    )(body)


def _k1_body(x_ref, g_ref, s_ref):
    i = pl.program_id(0)
    xt = x_ref[...]
    g = lax.dot_general(xt, xt, (((0,), (0,)), ((), ())),
                        preferred_element_type=f32)
    s = jnp.sum(xt, axis=0, keepdims=True)
    spad = jnp.concatenate([s, jnp.zeros((7, 128), f32)], axis=0)

    @pl.when(i == 0)
    def _():
        g_ref[...] = g
        s_ref[...] = spad

    @pl.when(i > 0)
    def _():
        g_ref[...] += g
        s_ref[...] += spad


def _k2_body(x_ref, w_ref, b_ref, h_ref):
    i = pl.program_id(0)
    rows = lax.broadcasted_iota(i32, (TR, 64), 0) + i * TR
    h = jnp.dot(x_ref[...], w_ref[...], preferred_element_type=f32) + b_ref[0:1, :]
    h_ref[...] = jnp.where(rows < NV, jnp.maximum(h, 0.0), 0.0)


def _k6_body(h2_ref, lap_ref, gv_ref, kid_ref, klap_ref, kew_ref, kns_ref,
             y_ref, st_ref):
    i = pl.program_id(0)
    gv = gv_ref[...]
    y = (jnp.dot(h2_ref[...], kid_ref[...], preferred_element_type=f32)
         + jnp.dot(lap_ref[...], klap_ref[...], preferred_element_type=f32)
         + jnp.dot(gv[:, :64], kew_ref[...], preferred_element_type=f32)
         + jnp.dot(gv[:, 64:], kns_ref[...], preferred_element_type=f32))
    y_ref[...] = y
    st = jnp.concatenate([jnp.sum(y, axis=0, keepdims=True),
                          jnp.sum(y * y, axis=0, keepdims=True),
                          jnp.zeros((6, 64), f32)], axis=0)

    @pl.when(i == 0)
    def _():
        st_ref[...] = st

    @pl.when(i > 0)
    def _():
        st_ref[...] += st


def _k7_body(y_ref, s2_ref, t2_ref, w3_ref, b3_ref, z_ref, st_ref):
    i = pl.program_id(0)
    h3 = jnp.maximum(y_ref[...] * s2_ref[0:1, :] + t2_ref[0:1, :], 0.0)
    z = jnp.dot(h3, w3_ref[...], preferred_element_type=f32) + b3_ref[0:1, :]
    rows = lax.broadcasted_iota(i32, (TR, 128), 0) + i * TR
    z = jnp.where(rows < NV, z, 0.0)
    z_ref[...] = z
    st = jnp.concatenate([jnp.sum(z, axis=0, keepdims=True),
                          jnp.sum(z * z, axis=0, keepdims=True),
                          jnp.zeros((6, 128), f32)], axis=0)

    @pl.when(i == 0)
    def _():
        st_ref[...] = st

    @pl.when(i > 0)
    def _():
        st_ref[...] += st


def _k8_body(z_ref, x_ref, s3_ref, t3_ref, o_ref):
    o_ref[...] = jnp.maximum(
        z_ref[...] * s3_ref[0:1, :] + t3_ref[0:1, :] + x_ref[...], 0.0)


def _row_spec(w):
    return pl.BlockSpec((TR, w), lambda i: (i, 0))


def _full_spec(h, w):
    return pl.BlockSpec((h, w), lambda i: (0, 0))


_GRID = NVP // TR


def _tc_call(body, in_specs, out_specs, out_shapes):
    return pl.pallas_call(
        body,
        grid=(_GRID,),
        in_specs=in_specs,
        out_specs=out_specs,
        out_shape=out_shapes,
        compiler_params=pltpu.CompilerParams(
            dimension_semantics=("arbitrary",)),
    )


def kernel(x, W1a, b1a, coeffs, W3a, b3a, g1a, be1a, g2a, be2a, g3a, be3a,
           G_rows, G_cols, G_vals, L_rows, L_cols, L_vals,
           F_rows, F_cols, F_vals, EW, NS_):
    N = B * NV
    eyeB = jnp.eye(B, dtype=f32)

    x_vm = jnp.transpose(x, (2, 0, 1)).reshape(NV, B * IN_CH)
    x_vm = jnp.pad(x_vm, ((0, NVP - NV), (0, 0)))

    g128, csum8 = _tc_call(
        _k1_body,
        [_row_spec(128)],
        [_full_spec(128, 128), _full_spec(8, 128)],
        [jax.ShapeDtypeStruct((128, 128), f32),
         jax.ShapeDtypeStruct((8, 128), f32)],
    )(x_vm)
    csum = csum8[0]

    mu_x = csum.reshape(B, IN_CH).sum(0) / N
    Sig = sum(g128[b * IN_CH:(b + 1) * IN_CH, b * IN_CH:(b + 1) * IN_CH]
              for b in range(B)) / N
    mu_h = W1a @ mu_x + b1a
    Eh2 = jnp.einsum('ci,ij,cj->c', W1a, Sig, W1a) + 2 * b1a * (W1a @ mu_x) + b1a ** 2
    s1 = g1a / jnp.sqrt(Eh2 - mu_h ** 2 + EPS)
    W1K = jnp.kron(eyeB, (W1a * s1[:, None]).T)
    b1K = jnp.tile(s1 * (b1a - mu_h) + be1a, B)
    b1K8 = jnp.tile(b1K[None, :], (8, 1))

    (h2,) = _tc_call(
        _k2_body,
        [_row_spec(128), _full_spec(128, 64), _full_spec(8, 64)],
        [_row_spec(64)],
        [jax.ShapeDtypeStruct((NVP, 64), f32)],
    )(x_vm, W1K, b1K8)

    Gc9 = G_cols.reshape(3, NF, 3)
    Gv9 = G_vals.reshape(3, NF, 3)
    C9 = jnp.transpose(Gc9, (1, 0, 2)).reshape(NF, 9)
    WE9 = jnp.transpose(Gv9 * EW.T[:, :, None], (1, 0, 2)).reshape(NF, 9)
    WN9 = jnp.transpose(Gv9 * NS_.T[:, :, None], (1, 0, 2)).reshape(NF, 9)
    WG = jnp.stack([WE9, WN9], axis=-1)
    idxG = C9.reshape(NF // CH, CH * 9)
    wG = WG.reshape(NF // CH, CH * 9 * 2)

    Lc7 = jnp.pad(L_cols.reshape(NV, 7), ((0, NVP - NV), (0, 0)))
    Lw7 = jnp.pad(L_vals.reshape(NV, 7), ((0, NVP - NV), (0, 0)))
    idxL = Lc7.reshape(NVP // CH, CH * 7)
    wL = Lw7.reshape(NVP // CH, CH * 7)

    Fc6 = jnp.pad(F_cols.reshape(NV, 6), ((0, NVP - NV), (0, 0)))
    Fw6 = jnp.pad(F_vals.reshape(NV, 6), ((0, NVP - NV), (0, 0)))
    idxF = Fc6.reshape(NVP // CH, CH * 6)
    wF = Fw6.reshape(NVP // CH, CH * 6)

    lap = _make_sc_gather(NVP, 64, NVP, 7, 1)(h2, idxL, wL)
    gf = _make_sc_gather(NVP, 64, NF, 9, 2)(h2, idxG, wG)
    gv = _make_sc_gather(NF, 128, NVP, 6, 1)(gf, idxF, wF)

    Ks = [jnp.kron(eyeB, coeffs[j::4, :]) for j in range(4)]
    y, st6 = _tc_call(
        _k6_body,
        [_row_spec(64), _row_spec(64), _row_spec(128),
         _full_spec(64, 64), _full_spec(64, 64), _full_spec(64, 64),
         _full_spec(64, 64)],
        [_row_spec(64), _full_spec(8, 64)],
        [jax.ShapeDtypeStruct((NVP, 64), f32),
         jax.ShapeDtypeStruct((8, 64), f32)],
    )(h2, lap, gv, Ks[0], Ks[1], Ks[2], Ks[3])

    mu_y = st6[0].reshape(B, NECK).sum(0) / N
    var_y = st6[1].reshape(B, NECK).sum(0) / N - mu_y ** 2
    s2 = g2a / jnp.sqrt(var_y + EPS)
    t2 = -mu_y * s2 + be2a
    s2c8 = jnp.tile(jnp.tile(s2, B)[None, :], (8, 1))
    t2c8 = jnp.tile(jnp.tile(t2, B)[None, :], (8, 1))

    W3K = jnp.kron(eyeB, W3a.T)
    b3K8 = jnp.tile(jnp.tile(b3a, B)[None, :], (8, 1))
    z, st7 = _tc_call(
        _k7_body,
        [_row_spec(64), _full_spec(8, 64), _full_spec(8, 64),
         _full_spec(64, 128), _full_spec(8, 128)],
        [_row_spec(128), _full_spec(8, 128)],
        [jax.ShapeDtypeStruct((NVP, 128), f32),
         jax.ShapeDtypeStruct((8, 128), f32)],
    )(y, s2c8, t2c8, W3K, b3K8)

    mu_z = st7[0].reshape(B, OUT_CH).sum(0) / N
    var_z = st7[1].reshape(B, OUT_CH).sum(0) / N - mu_z ** 2
    s3 = g3a / jnp.sqrt(var_z + EPS)
    t3 = -mu_z * s3 + be3a
    s3c8 = jnp.tile(jnp.tile(s3, B)[None, :], (8, 1))
    t3c8 = jnp.tile(jnp.tile(t3, B)[None, :], (8, 1))

    (out_vm,) = _tc_call(
        _k8_body,
        [_row_spec(128), _row_spec(128), _full_spec(8, 128), _full_spec(8, 128)],
        [_row_spec(128)],
        [jax.ShapeDtypeStruct((NVP, 128), f32)],
    )(z, x_vm, s3c8, t3c8)

    return jnp.transpose(out_vm[:NV].reshape(NV, B, OUT_CH), (1, 2, 0))


# speedup vs baseline: 67.8497x; 1.0815x over previous
import functools

import jax
import jax.numpy as jnp
from jax import lax
from jax.experimental import pallas as pl
from jax.experimental.pallas import tpu as pltpu
from jax.experimental.pallas import tpu_sc as plsc

NV = 40962
NF = 81920
B = 4
IN_CH = 32
NECK = 16
OUT_CH = 32
EPS = 1e-5

NVP = 43008
TR = 1024
CH = 8
OT = 64
NC = 2
NS = 16
NW = NC * NS

f32 = jnp.float32
i32 = jnp.int32


def _make_sc_gather(t_rows, din, r_rows, k_nnz, n_w):
    dout = din * n_w
    rows_pw = r_rows // NW
    chunks_pw = rows_pw // CH
    tiles_pw = rows_pw // OT
    chunks_pt = OT // CH
    mesh = plsc.VectorSubcoreMesh(core_axis_name="c", subcore_axis_name="s")

    def body(table, idxh, wh, outh, idx_v, w_v, rows_v, out_v, sem0, sem1):
        cid = lax.axis_index("c")
        sid = lax.axis_index("s")
        wid = sid * NC + cid
        cbase = wid * chunks_pw
        pltpu.sync_copy(idxh.at[pl.ds(cbase, chunks_pw)], idx_v)
        pltpu.sync_copy(wh.at[pl.ds(cbase, chunks_pw)], w_v)

        sems = [sem0, sem1]

        def start(cl, b):
            pltpu.async_copy(table.at[idx_v.at[cl]], rows_v.at[b], sems[b])

        def wait(cl, b):
            pltpu.make_async_copy(table.at[idx_v.at[cl]], rows_v.at[b],
                                  sems[b]).wait()

        rowlen = CH * k_nnz * n_w
        offs = list(range(0, max(rowlen - 15, 1), 16))
        if rowlen % 16:
            offs.append(rowlen - 16)
        nv = din // 16

        start(0, 0)
        start(1, 1)

        def pair_body(p, _):
            for b in range(2):
                cl = p * 2 + b
                wait(cl, b)
                wvecs = [w_v[cl, pl.ds(o, 16)] for o in offs]

                def wscal(j):
                    if j >= offs[-1]:
                        return wvecs[-1][j - offs[-1]]
                    return wvecs[j // 16][j % 16]

                orow = lax.rem(cl, chunks_pt) * CH
                for r in range(CH):
                    accs = [[None] * nv for _ in range(n_w)]
                    for kk in range(k_nnz):
                        ws = [wscal((r * k_nnz + kk) * n_w + w)
                              for w in range(n_w)]
                        for v in range(nv):
                            rv = rows_v[b, r * k_nnz + kk, pl.ds(v * 16, 16)]
                            for w in range(n_w):
                                pr = ws[w] * rv
                                accs[w][v] = pr if kk == 0 else accs[w][v] + pr
                    for w in range(n_w):
                        for v in range(nv):
                            out_v[orow + r,
                                  pl.ds(w * din + v * 16, 16)] = accs[w][v]

                @pl.when(cl + 2 < chunks_pw)
                def _():
                    start(cl + 2, b)

                @pl.when(lax.rem(cl, chunks_pt) == chunks_pt - 1)
                def _():
                    t = lax.div(cl, chunks_pt)
                    pltpu.sync_copy(
                        out_v, outh.at[pl.ds(wid * rows_pw + t * OT, OT)])
            return 0

        lax.fori_loop(0, chunks_pw // 2, pair_body, 0)

    return functools.partial(
        pl.kernel,
        out_type=jax.ShapeDtypeStruct((r_rows, dout), f32),
        mesh=mesh,
        scratch_types=[
            pltpu.VMEM((chunks_pw, CH * k_nnz), i32),
            pltpu.VMEM((chunks_pw, CH * k_nnz * n_w), f32),
            pltpu.VMEM((2, CH * k_nnz, din), f32),
            pltpu.VMEM((OT, dout), f32),
            pltpu.SemaphoreType.DMA,
            pltpu.SemaphoreType.DMA,
        ],
        compiler_params=pltpu.CompilerParams(use_tc_tiling_on_sc=False),
    )(body)


def _k1_body(x_ref, g_ref, s_ref):
    i = pl.program_id(0)
    xt = x_ref[...]
    g = lax.dot_general(xt, xt, (((0,), (0,)), ((), ())),
                        preferred_element_type=f32)
    s = jnp.sum(xt, axis=0, keepdims=True)
    spad = jnp.concatenate([s, jnp.zeros((7, 128), f32)], axis=0)

    @pl.when(i == 0)
    def _():
        g_ref[...] = g
        s_ref[...] = spad

    @pl.when(i > 0)
    def _():
        g_ref[...] += g
        s_ref[...] += spad


def _k2_body(x_ref, w_ref, b_ref, h_ref):
    i = pl.program_id(0)
    rows = lax.broadcasted_iota(i32, (TR, 64), 0) + i * TR
    h = jnp.dot(x_ref[...], w_ref[...], preferred_element_type=f32) + b_ref[0:1, :]
    h_ref[...] = jnp.where(rows < NV, jnp.maximum(h, 0.0), 0.0)


def _k6_body(h2_ref, lap_ref, gv_ref, kid_ref, klap_ref, kew_ref, kns_ref,
             y_ref, st_ref):
    i = pl.program_id(0)
    gv = gv_ref[...]
    y = (jnp.dot(h2_ref[...], kid_ref[...], preferred_element_type=f32)
         + jnp.dot(lap_ref[...], klap_ref[...], preferred_element_type=f32)
         + jnp.dot(gv[:, :64], kew_ref[...], preferred_element_type=f32)
         + jnp.dot(gv[:, 64:], kns_ref[...], preferred_element_type=f32))
    y_ref[...] = y
    st = jnp.concatenate([jnp.sum(y, axis=0, keepdims=True),
                          jnp.sum(y * y, axis=0, keepdims=True),
                          jnp.zeros((6, 64), f32)], axis=0)

    @pl.when(i == 0)
    def _():
        st_ref[...] = st

    @pl.when(i > 0)
    def _():
        st_ref[...] += st


def _k7_body(y_ref, s2_ref, t2_ref, w3_ref, b3_ref, z_ref, st_ref):
    i = pl.program_id(0)
    h3 = jnp.maximum(y_ref[...] * s2_ref[0:1, :] + t2_ref[0:1, :], 0.0)
    z = jnp.dot(h3, w3_ref[...], preferred_element_type=f32) + b3_ref[0:1, :]
    rows = lax.broadcasted_iota(i32, (TR, 128), 0) + i * TR
    z = jnp.where(rows < NV, z, 0.0)
    z_ref[...] = z
    st = jnp.concatenate([jnp.sum(z, axis=0, keepdims=True),
                          jnp.sum(z * z, axis=0, keepdims=True),
                          jnp.zeros((6, 128), f32)], axis=0)

    @pl.when(i == 0)
    def _():
        st_ref[...] = st

    @pl.when(i > 0)
    def _():
        st_ref[...] += st


def _k8_body(z_ref, x_ref, s3_ref, t3_ref, o_ref):
    o_ref[...] = jnp.maximum(
        z_ref[...] * s3_ref[0:1, :] + t3_ref[0:1, :] + x_ref[...], 0.0)


def _row_spec(w):
    return pl.BlockSpec((TR, w), lambda i: (i, 0))


def _full_spec(h, w):
    return pl.BlockSpec((h, w), lambda i: (0, 0))


_GRID = NVP // TR


def _tc_call(body, in_specs, out_specs, out_shapes):
    return pl.pallas_call(
        body,
        grid=(_GRID,),
        in_specs=in_specs,
        out_specs=out_specs,
        out_shape=out_shapes,
        compiler_params=pltpu.CompilerParams(
            dimension_semantics=("arbitrary",)),
    )


def kernel(x, W1a, b1a, coeffs, W3a, b3a, g1a, be1a, g2a, be2a, g3a, be3a,
           G_rows, G_cols, G_vals, L_rows, L_cols, L_vals,
           F_rows, F_cols, F_vals, EW, NS_):
    N = B * NV
    eyeB = jnp.eye(B, dtype=f32)

    x_vm = jnp.transpose(x, (2, 0, 1)).reshape(NV, B * IN_CH)
    x_vm = jnp.pad(x_vm, ((0, NVP - NV), (0, 0)))

    g128, csum8 = _tc_call(
        _k1_body,
        [_row_spec(128)],
        [_full_spec(128, 128), _full_spec(8, 128)],
        [jax.ShapeDtypeStruct((128, 128), f32),
         jax.ShapeDtypeStruct((8, 128), f32)],
    )(x_vm)
    csum = csum8[0]

    mu_x = csum.reshape(B, IN_CH).sum(0) / N
    Sig = sum(g128[b * IN_CH:(b + 1) * IN_CH, b * IN_CH:(b + 1) * IN_CH]
              for b in range(B)) / N
    mu_h = W1a @ mu_x + b1a
    Eh2 = jnp.einsum('ci,ij,cj->c', W1a, Sig, W1a) + 2 * b1a * (W1a @ mu_x) + b1a ** 2
    s1 = g1a / jnp.sqrt(Eh2 - mu_h ** 2 + EPS)
    W1K = jnp.kron(eyeB, (W1a * s1[:, None]).T)
    b1K = jnp.tile(s1 * (b1a - mu_h) + be1a, B)
    b1K8 = jnp.tile(b1K[None, :], (8, 1))

    (h2,) = _tc_call(
        _k2_body,
        [_row_spec(128), _full_spec(128, 64), _full_spec(8, 64)],
        [_row_spec(64)],
        [jax.ShapeDtypeStruct((NVP, 64), f32)],
    )(x_vm, W1K, b1K8)

    Gc9 = G_cols.reshape(3, NF, 3)
    Gv9 = G_vals.reshape(3, NF, 3)
    C9 = jnp.transpose(Gc9, (1, 0, 2)).reshape(NF, 9)
    WE9 = jnp.transpose(Gv9 * EW.T[:, :, None], (1, 0, 2)).reshape(NF, 9)
    WN9 = jnp.transpose(Gv9 * NS_.T[:, :, None], (1, 0, 2)).reshape(NF, 9)
    WG = jnp.stack([WE9, WN9], axis=-1)
    idxG = C9.reshape(NF // CH, CH * 9)
    wG = WG.reshape(NF // CH, CH * 9 * 2)

    Lc7 = jnp.pad(L_cols.reshape(NV, 7), ((0, NVP - NV), (0, 0)))
    Lw7 = jnp.pad(L_vals.reshape(NV, 7), ((0, NVP - NV), (0, 0)))
    idxL = Lc7.reshape(NVP // CH, CH * 7)
    wL = Lw7.reshape(NVP // CH, CH * 7)

    Fc6 = jnp.pad(F_cols.reshape(NV, 6), ((0, NVP - NV), (0, 0)))
    Fw6 = jnp.pad(F_vals.reshape(NV, 6), ((0, NVP - NV), (0, 0)))
    idxF = Fc6.reshape(NVP // CH, CH * 6)
    wF = Fw6.reshape(NVP // CH, CH * 6)

    lap = _make_sc_gather(NVP, 64, NVP, 7, 1)(h2, idxL, wL)
    gf = _make_sc_gather(NVP, 64, NF, 9, 2)(h2, idxG, wG)
    gv = _make_sc_gather(NF, 128, NVP, 6, 1)(gf, idxF, wF)

    Ks = [jnp.kron(eyeB, coeffs[j::4, :]) for j in range(4)]
    y, st6 = _tc_call(
        _k6_body,
        [_row_spec(64), _row_spec(64), _row_spec(128),
         _full_spec(64, 64), _full_spec(64, 64), _full_spec(64, 64),
         _full_spec(64, 64)],
        [_row_spec(64), _full_spec(8, 64)],
        [jax.ShapeDtypeStruct((NVP, 64), f32),
         jax.ShapeDtypeStruct((8, 64), f32)],
    )(h2, lap, gv, Ks[0], Ks[1], Ks[2], Ks[3])

    mu_y = st6[0].reshape(B, NECK).sum(0) / N
    var_y = st6[1].reshape(B, NECK).sum(0) / N - mu_y ** 2
    s2 = g2a / jnp.sqrt(var_y + EPS)
    t2 = -mu_y * s2 + be2a
    s2c8 = jnp.tile(jnp.tile(s2, B)[None, :], (8, 1))
    t2c8 = jnp.tile(jnp.tile(t2, B)[None, :], (8, 1))

    W3K = jnp.kron(eyeB, W3a.T)
    b3K8 = jnp.tile(jnp.tile(b3a, B)[None, :], (8, 1))
    z, st7 = _tc_call(
        _k7_body,
        [_row_spec(64), _full_spec(8, 64), _full_spec(8, 64),
         _full_spec(64, 128), _full_spec(8, 128)],
        [_row_spec(128), _full_spec(8, 128)],
        [jax.ShapeDtypeStruct((NVP, 128), f32),
         jax.ShapeDtypeStruct((8, 128), f32)],
    )(y, s2c8, t2c8, W3K, b3K8)

    mu_z = st7[0].reshape(B, OUT_CH).sum(0) / N
    var_z = st7[1].reshape(B, OUT_CH).sum(0) / N - mu_z ** 2
    s3 = g3a / jnp.sqrt(var_z + EPS)
    t3 = -mu_z * s3 + be3a
    s3c8 = jnp.tile(jnp.tile(s3, B)[None, :], (8, 1))
    t3c8 = jnp.tile(jnp.tile(t3, B)[None, :], (8, 1))

    (out_vm,) = _tc_call(
        _k8_body,
        [_row_spec(128), _row_spec(128), _full_spec(8, 128), _full_spec(8, 128)],
        [_row_spec(128)],
        [jax.ShapeDtypeStruct((NVP, 128), f32)],
    )(z, x_vm, s3c8, t3c8)

    return jnp.transpose(out_vm[:NV].reshape(NV, B, OUT_CH), (1, 2, 0))


# speedup vs baseline: 70.2640x; 1.0356x over previous
import functools

import jax
import jax.numpy as jnp
from jax import lax
from jax.experimental import pallas as pl
from jax.experimental.pallas import tpu as pltpu
from jax.experimental.pallas import tpu_sc as plsc

NV = 40962
NF = 81920
B = 4
IN_CH = 32
NECK = 16
OUT_CH = 32
EPS = 1e-5

NVP = 43008
TR = 1024
CH = 8
OT = 64
NC = 2
NS = 16
NW = NC * NS
NBUF = 4

f32 = jnp.float32
i32 = jnp.int32


def _make_sc_gather(t_rows, din, r_rows, k_nnz, n_w):
    dout = din * n_w
    rows_pw = r_rows // NW
    chunks_pw = rows_pw // CH
    tiles_pw = rows_pw // OT
    chunks_pt = OT // CH
    mesh = plsc.VectorSubcoreMesh(core_axis_name="c", subcore_axis_name="s")

    def body(table, idxh, wh, outh, idx_v, w_v, rows_v, out_v, *sems):
        cid = lax.axis_index("c")
        sid = lax.axis_index("s")
        wid = sid * NC + cid
        cbase = wid * chunks_pw
        pltpu.sync_copy(idxh.at[pl.ds(cbase, chunks_pw)], idx_v)
        pltpu.sync_copy(wh.at[pl.ds(cbase, chunks_pw)], w_v)

        def start(cl, b):
            pltpu.async_copy(table.at[idx_v.at[cl]], rows_v.at[b], sems[b])

        def wait(cl, b):
            pltpu.make_async_copy(table.at[idx_v.at[cl]], rows_v.at[b],
                                  sems[b]).wait()

        rowlen = CH * k_nnz * n_w
        offs = list(range(0, max(rowlen - 15, 1), 16))
        if rowlen % 16:
            offs.append(rowlen - 16)
        nv = din // 16

        for b in range(NBUF):
            start(b, b)

        def pair_body(p, _):
            for b in range(NBUF):
                cl = p * NBUF + b
                wait(cl, b)
                wvecs = [w_v[cl, pl.ds(o, 16)] for o in offs]

                def wscal(j):
                    if j >= offs[-1]:
                        return wvecs[-1][j - offs[-1]]
                    return wvecs[j // 16][j % 16]

                orow = lax.rem(cl, chunks_pt) * CH
                for r in range(CH):
                    accs = [[None] * nv for _ in range(n_w)]
                    for kk in range(k_nnz):
                        ws = [wscal((r * k_nnz + kk) * n_w + w)
                              for w in range(n_w)]
                        for v in range(nv):
                            rv = rows_v[b, r * k_nnz + kk, pl.ds(v * 16, 16)]
                            for w in range(n_w):
                                pr = ws[w] * rv
                                accs[w][v] = pr if kk == 0 else accs[w][v] + pr
                    for w in range(n_w):
                        for v in range(nv):
                            out_v[orow + r,
                                  pl.ds(w * din + v * 16, 16)] = accs[w][v]

                @pl.when(cl + NBUF < chunks_pw)
                def _():
                    start(cl + NBUF, b)

                @pl.when(lax.rem(cl, chunks_pt) == chunks_pt - 1)
                def _():
                    t = lax.div(cl, chunks_pt)
                    pltpu.sync_copy(
                        out_v, outh.at[pl.ds(wid * rows_pw + t * OT, OT)])
            return 0

        lax.fori_loop(0, chunks_pw // NBUF, pair_body, 0)

    return functools.partial(
        pl.kernel,
        out_type=jax.ShapeDtypeStruct((r_rows, dout), f32),
        mesh=mesh,
        scratch_types=[
            pltpu.VMEM((chunks_pw, CH * k_nnz), i32),
            pltpu.VMEM((chunks_pw, CH * k_nnz * n_w), f32),
            pltpu.VMEM((NBUF, CH * k_nnz, din), f32),
            pltpu.VMEM((OT, dout), f32),
        ] + [pltpu.SemaphoreType.DMA] * NBUF,
        compiler_params=pltpu.CompilerParams(use_tc_tiling_on_sc=False),
    )(body)


def _k1_body(x_ref, g_ref, s_ref):
    i = pl.program_id(0)
    xt = jnp.transpose(x_ref[...].reshape(128, TR))
    rows = lax.broadcasted_iota(i32, (TR, 128), 0) + i * TR
    xt = jnp.where(rows < NV, xt, 0.0)
    g = lax.dot_general(xt, xt, (((0,), (0,)), ((), ())),
                        preferred_element_type=f32)
    s = jnp.sum(xt, axis=0, keepdims=True)
    spad = jnp.concatenate([s, jnp.zeros((7, 128), f32)], axis=0)

    @pl.when(i == 0)
    def _():
        g_ref[...] = g
        s_ref[...] = spad

    @pl.when(i > 0)
    def _():
        g_ref[...] += g
        s_ref[...] += spad


def _k2_body(x_ref, w_ref, b_ref, h_ref):
    i = pl.program_id(0)
    xt = jnp.transpose(x_ref[...].reshape(128, TR))
    rows = lax.broadcasted_iota(i32, (TR, 64), 0) + i * TR
    h = jnp.dot(xt, w_ref[...], preferred_element_type=f32) + b_ref[0:1, :]
    h_ref[...] = jnp.where(rows < NV, jnp.maximum(h, 0.0), 0.0)


def _k6_body(h2_ref, lap_ref, gv_ref, kid_ref, klap_ref, kew_ref, kns_ref,
             y_ref, st_ref):
    i = pl.program_id(0)
    gv = gv_ref[...]
    y = (jnp.dot(h2_ref[...], kid_ref[...], preferred_element_type=f32)
         + jnp.dot(lap_ref[...], klap_ref[...], preferred_element_type=f32)
         + jnp.dot(gv[:, :64], kew_ref[...], preferred_element_type=f32)
         + jnp.dot(gv[:, 64:], kns_ref[...], preferred_element_type=f32))
    y_ref[...] = y
    st = jnp.concatenate([jnp.sum(y, axis=0, keepdims=True),
                          jnp.sum(y * y, axis=0, keepdims=True),
                          jnp.zeros((6, 64), f32)], axis=0)

    @pl.when(i == 0)
    def _():
        st_ref[...] = st

    @pl.when(i > 0)
    def _():
        st_ref[...] += st


def _k7_body(y_ref, s2_ref, t2_ref, w3_ref, b3_ref, z_ref, st_ref):
    i = pl.program_id(0)
    h3 = jnp.maximum(y_ref[...] * s2_ref[0:1, :] + t2_ref[0:1, :], 0.0)
    z = jnp.dot(h3, w3_ref[...], preferred_element_type=f32) + b3_ref[0:1, :]
    rows = lax.broadcasted_iota(i32, (TR, 128), 0) + i * TR
    z = jnp.where(rows < NV, z, 0.0)
    z_ref[...] = z
    st = jnp.concatenate([jnp.sum(z, axis=0, keepdims=True),
                          jnp.sum(z * z, axis=0, keepdims=True),
                          jnp.zeros((6, 128), f32)], axis=0)

    @pl.when(i == 0)
    def _():
        st_ref[...] = st

    @pl.when(i > 0)
    def _():
        st_ref[...] += st


def _k8_body(z_ref, x_ref, s3_ref, t3_ref, o_ref):
    xt = jnp.transpose(x_ref[...].reshape(128, TR))
    o = jnp.maximum(z_ref[...] * s3_ref[0:1, :] + t3_ref[0:1, :] + xt, 0.0)
    o_ref[...] = jnp.transpose(o).reshape(4, 32, TR)


def _row_spec(w):
    return pl.BlockSpec((TR, w), lambda i: (i, 0))


def _full_spec(h, w):
    return pl.BlockSpec((h, w), lambda i: (0, 0))


_GRID = NVP // TR
_GRIDX = -(-NV // TR)


def _tc_call(body, in_specs, out_specs, out_shapes, grid=_GRID):
    return pl.pallas_call(
        body,
        grid=(grid,),
        in_specs=in_specs,
        out_specs=out_specs,
        out_shape=out_shapes,
        compiler_params=pltpu.CompilerParams(
            dimension_semantics=("arbitrary",)),
    )


def _x_spec(clamp=None):
    if clamp is None:
        return pl.BlockSpec((B, IN_CH, TR), lambda i: (0, 0, i))
    return pl.BlockSpec((B, IN_CH, TR),
                        lambda i: (0, 0, jnp.minimum(i, clamp)))


def kernel(x, W1a, b1a, coeffs, W3a, b3a, g1a, be1a, g2a, be2a, g3a, be3a,
           G_rows, G_cols, G_vals, L_rows, L_cols, L_vals,
           F_rows, F_cols, F_vals, EW, NS_):
    N = B * NV
    eyeB = jnp.eye(B, dtype=f32)

    g128, csum8 = _tc_call(
        _k1_body,
        [_x_spec()],
        [_full_spec(128, 128), _full_spec(8, 128)],
        [jax.ShapeDtypeStruct((128, 128), f32),
         jax.ShapeDtypeStruct((8, 128), f32)],
        grid=_GRIDX,
    )(x)
    csum = csum8[0]

    mu_x = csum.reshape(B, IN_CH).sum(0) / N
    Sig = sum(g128[b * IN_CH:(b + 1) * IN_CH, b * IN_CH:(b + 1) * IN_CH]
              for b in range(B)) / N
    mu_h = W1a @ mu_x + b1a
    Eh2 = jnp.einsum('ci,ij,cj->c', W1a, Sig, W1a) + 2 * b1a * (W1a @ mu_x) + b1a ** 2
    s1 = g1a / jnp.sqrt(Eh2 - mu_h ** 2 + EPS)
    W1K = jnp.kron(eyeB, (W1a * s1[:, None]).T)
    b1K = jnp.tile(s1 * (b1a - mu_h) + be1a, B)
    b1K8 = jnp.tile(b1K[None, :], (8, 1))

    (h2,) = _tc_call(
        _k2_body,
        [_x_spec(clamp=_GRIDX - 1), _full_spec(128, 64), _full_spec(8, 64)],
        [_row_spec(64)],
        [jax.ShapeDtypeStruct((NVP, 64), f32)],
    )(x, W1K, b1K8)

    Gc9 = G_cols.reshape(3, NF, 3)
    Gv9 = G_vals.reshape(3, NF, 3)
    C9 = jnp.transpose(Gc9, (1, 0, 2)).reshape(NF, 9)
    WE9 = jnp.transpose(Gv9 * EW.T[:, :, None], (1, 0, 2)).reshape(NF, 9)
    WN9 = jnp.transpose(Gv9 * NS_.T[:, :, None], (1, 0, 2)).reshape(NF, 9)
    WG = jnp.stack([WE9, WN9], axis=-1)
    idxG = C9.reshape(NF // CH, CH * 9)
    wG = WG.reshape(NF // CH, CH * 9 * 2)

    Lc7 = jnp.pad(L_cols.reshape(NV, 7), ((0, NVP - NV), (0, 0)))
    Lw7 = jnp.pad(L_vals.reshape(NV, 7), ((0, NVP - NV), (0, 0)))
    idxL = Lc7.reshape(NVP // CH, CH * 7)
    wL = Lw7.reshape(NVP // CH, CH * 7)

    Fc6 = jnp.pad(F_cols.reshape(NV, 6), ((0, NVP - NV), (0, 0)))
    Fw6 = jnp.pad(F_vals.reshape(NV, 6), ((0, NVP - NV), (0, 0)))
    idxF = Fc6.reshape(NVP // CH, CH * 6)
    wF = Fw6.reshape(NVP // CH, CH * 6)

    lap = _make_sc_gather(NVP, 64, NVP, 7, 1)(h2, idxL, wL)
    gf = _make_sc_gather(NVP, 64, NF, 9, 2)(h2, idxG, wG)
    gv = _make_sc_gather(NF, 128, NVP, 6, 1)(gf, idxF, wF)

    Ks = [jnp.kron(eyeB, coeffs[j::4, :]) for j in range(4)]
    y, st6 = _tc_call(
        _k6_body,
        [_row_spec(64), _row_spec(64), _row_spec(128),
         _full_spec(64, 64), _full_spec(64, 64), _full_spec(64, 64),
         _full_spec(64, 64)],
        [_row_spec(64), _full_spec(8, 64)],
        [jax.ShapeDtypeStruct((NVP, 64), f32),
         jax.ShapeDtypeStruct((8, 64), f32)],
    )(h2, lap, gv, Ks[0], Ks[1], Ks[2], Ks[3])

    mu_y = st6[0].reshape(B, NECK).sum(0) / N
    var_y = st6[1].reshape(B, NECK).sum(0) / N - mu_y ** 2
    s2 = g2a / jnp.sqrt(var_y + EPS)
    t2 = -mu_y * s2 + be2a
    s2c8 = jnp.tile(jnp.tile(s2, B)[None, :], (8, 1))
    t2c8 = jnp.tile(jnp.tile(t2, B)[None, :], (8, 1))

    W3K = jnp.kron(eyeB, W3a.T)
    b3K8 = jnp.tile(jnp.tile(b3a, B)[None, :], (8, 1))
    z, st7 = _tc_call(
        _k7_body,
        [_row_spec(64), _full_spec(8, 64), _full_spec(8, 64),
         _full_spec(64, 128), _full_spec(8, 128)],
        [_row_spec(128), _full_spec(8, 128)],
        [jax.ShapeDtypeStruct((NVP, 128), f32),
         jax.ShapeDtypeStruct((8, 128), f32)],
    )(y, s2c8, t2c8, W3K, b3K8)

    mu_z = st7[0].reshape(B, OUT_CH).sum(0) / N
    var_z = st7[1].reshape(B, OUT_CH).sum(0) / N - mu_z ** 2
    s3 = g3a / jnp.sqrt(var_z + EPS)
    t3 = -mu_z * s3 + be3a
    s3c8 = jnp.tile(jnp.tile(s3, B)[None, :], (8, 1))
    t3c8 = jnp.tile(jnp.tile(t3, B)[None, :], (8, 1))

    (out,) = _tc_call(
        _k8_body,
        [_row_spec(128), _x_spec(), _full_spec(8, 128), _full_spec(8, 128)],
        [pl.BlockSpec((B, OUT_CH, TR), lambda i: (0, 0, i))],
        [jax.ShapeDtypeStruct((B, OUT_CH, NV), f32)],
        grid=_GRIDX,
    )(z, x, s3c8, t3c8)

    return out
